# baseline (device time: 142418 ns/iter reference)
import jax
import jax.numpy as jnp
from jax import lax
from jax.experimental import pallas as pl
from jax.experimental.pallas import tpu as pltpu

N_DEV = 8


def kernel(x, dest):
    rows, cols = x.shape
    n = N_DEV

    dest = dest.astype(jnp.int32)
    onehot = dest[:, None] == jnp.arange(n, dtype=jnp.int32)[None, :]
    cnt = jnp.sum(onehot, axis=0, dtype=jnp.int32)
    incl = jnp.cumsum(onehot, axis=0, dtype=jnp.int32)
    off = jnp.take_along_axis(incl, dest[:, None], axis=1)[:, 0] - 1

    def body(x_ref, dest_ref, off_ref, cnt_ref, out_ref,
             base_ref, count_sems, send_sem, recv_sem):
        me = lax.axis_index("i")

        barrier = pltpu.get_barrier_semaphore()
        for p in range(n):
            pl.semaphore_signal(
                barrier, 1, device_id=p,
                device_id_type=pl.DeviceIdType.LOGICAL)
        pl.semaphore_wait(barrier, n)

        for p in range(n):
            for d in range(n):
                pl.semaphore_signal(
                    count_sems.at[me, d], cnt_ref[d] + 1,
                    device_id=p, device_id_type=pl.DeviceIdType.LOGICAL)

        for d in range(n):
            base_ref[d] = 0
        for s in range(n):
            for d in range(n):
                sem = count_sems.at[s, d]
                pl.semaphore_wait(sem, 1)
                v = pl.semaphore_read(sem)

                @pl.when(v > 0)
                def _():
                    pl.semaphore_wait(sem, v)

                base_ref[d] = base_ref[d] + jnp.where(s < me, v, 0)

        def send_one(i, carry):
            d = dest_ref[i]
            dst_row = base_ref[d] + off_ref[i]
            rdma = pltpu.make_async_remote_copy(
                x_ref.at[pl.ds(i, 1)],
                out_ref.at[pl.ds(dst_row, 1)],
                send_sem, recv_sem,
                device_id=d, device_id_type=pl.DeviceIdType.LOGICAL)
            rdma.start()
            return carry

        lax.fori_loop(0, rows, send_one, 0)

        def wait_send_one(i, carry):
            desc = pltpu.make_async_remote_copy(
                x_ref.at[pl.ds(0, 1)], out_ref.at[pl.ds(0, 1)],
                send_sem, recv_sem,
                device_id=me, device_id_type=pl.DeviceIdType.LOGICAL)
            desc.wait_send()
            return carry

        lax.fori_loop(0, rows, wait_send_one, 0)

        def wait_recv_one(i, carry):
            desc = pltpu.make_async_remote_copy(
                x_ref.at[pl.ds(0, 1)], out_ref.at[pl.ds(0, 1)],
                send_sem, recv_sem,
                device_id=me, device_id_type=pl.DeviceIdType.LOGICAL)
            desc.wait_recv()
            return carry

        lax.fori_loop(0, rows, wait_recv_one, 0)

    return pl.pallas_call(
        body,
        out_shape=jax.ShapeDtypeStruct((rows, cols), x.dtype),
        in_specs=[
            pl.BlockSpec(memory_space=pltpu.VMEM),
            pl.BlockSpec(memory_space=pltpu.SMEM),
            pl.BlockSpec(memory_space=pltpu.SMEM),
            pl.BlockSpec(memory_space=pltpu.SMEM),
        ],
        out_specs=pl.BlockSpec(memory_space=pltpu.VMEM),
        scratch_shapes=[
            pltpu.SMEM((n,), jnp.int32),
            pltpu.SemaphoreType.REGULAR((n, n)),
            pltpu.SemaphoreType.DMA,
            pltpu.SemaphoreType.DMA,
        ],
        compiler_params=pltpu.CompilerParams(collective_id=0),
    )(x, dest, off, cnt)
